# Initial kernel scaffold; baseline (speedup 1.0000x reference)
import jax
import jax.numpy as jnp
from jax.experimental import pallas as pl


def kernel(x, adj, w, affine_weight, affine_bias, weight, bias):
    raise NotImplementedError("write your pallas kernel here")


# speedup vs baseline: 1.0504x; 1.0504x over previous
import jax
import jax.numpy as jnp
import numpy as np
from jax.experimental import pallas as pl
from jax.experimental.pallas import tpu as pltpu

_B, _P, _CIN, _COUT, _WDIM = 2, 10000, 128, 128, 512
_RB = 400
_SQRT2 = np.float32(np.sqrt(2.0))


def _fused_kernel(x_ref, adj_ref, w_ref, aw_ref, ab_ref, wt_ref, b_ref,
                  out_ref, y_ref):
    i = pl.program_id(0)

    @pl.when(i == 0)
    def _compute_y():
        styles = jax.lax.dot_general(
            w_ref[:], aw_ref[:], (((1,), (1,)), ((), ())),
            preferred_element_type=jnp.float32) * np.float32(1.0 / np.sqrt(_WDIM))
        styles = styles + ab_ref[:]
        for b in range(_B):
            ww = wt_ref[:] * styles[b][None, :]
            d = jax.lax.rsqrt(jnp.sum(ww * ww, axis=1) + 1e-8)
            ww = ww * d[:, None]
            yb = jax.lax.dot_general(
                x_ref[b], ww, (((1,), (1,)), ((), ())),
                preferred_element_type=jnp.float32)
            y_ref[:, b * _COUT:(b + 1) * _COUT] = yb.astype(jnp.bfloat16)

    acc = jax.lax.dot_general(
        adj_ref[:].astype(jnp.bfloat16), y_ref[:],
        (((1,), (0,)), ((), ())),
        preferred_element_type=jnp.float32)
    for b in range(_B):
        v = acc[:, b * _COUT:(b + 1) * _COUT] + b_ref[:]
        out_ref[b] = jnp.where(v >= 0, v, 0.2 * v) * _SQRT2


def kernel(x, adj, w, affine_weight, affine_bias, weight, bias):
    grid = (_P // _RB,)
    return pl.pallas_call(
        _fused_kernel,
        grid=grid,
        in_specs=[
            pl.BlockSpec((_B, _P, _CIN), lambda i: (0, 0, 0)),
            pl.BlockSpec((_RB, _P), lambda i: (i, 0)),
            pl.BlockSpec((_B, _WDIM), lambda i: (0, 0)),
            pl.BlockSpec((_CIN, _WDIM), lambda i: (0, 0)),
            pl.BlockSpec((1, _CIN), lambda i: (0, 0)),
            pl.BlockSpec((_COUT, _CIN), lambda i: (0, 0)),
            pl.BlockSpec((1, _COUT), lambda i: (0, 0)),
        ],
        out_specs=pl.BlockSpec((_B, _RB, _COUT), lambda i: (0, i, 0)),
        out_shape=jax.ShapeDtypeStruct((_B, _P, _COUT), jnp.float32),
        scratch_shapes=[pltpu.VMEM((_P, _B * _COUT), jnp.bfloat16)],
        compiler_params=pltpu.CompilerParams(
            dimension_semantics=("arbitrary",)),
    )(x, adj, w, affine_weight, affine_bias.reshape(1, _CIN),
      weight, bias.reshape(1, _COUT))
